# baseline (speedup 1.0000x reference)
import jax
import jax.numpy as jnp
from jax import lax
from jax.experimental import pallas as pl
from jax.experimental.pallas import tpu as pltpu
from jax.experimental.pallas import tpu_sc as plsc

N_TOKENS = 32768
N_SEGS = 16
NUM_SUBCORES = 16
CHUNK = N_TOKENS // NUM_SUBCORES
LANES = 16
I32_MAX = jnp.iinfo(jnp.int32).max


def _sc_body(values_hbm, ps_hbm, stage_hbm, out_hbm,
             vals_v, ps_v, res_v, buf_v, out_v):
    sid = lax.axis_index("s")
    base = sid * CHUNK

    pltpu.sync_copy(values_hbm.at[pl.ds(base, CHUNK)], vals_v)
    pltpu.sync_copy(ps_hbm, ps_v)

    iota = lax.iota(jnp.int32, LANES)
    iota_f = iota.astype(jnp.float32)
    neg_inf = jnp.float32(-jnp.inf)
    pos_inf = jnp.float32(jnp.inf)
    neg16 = jnp.full((LANES,), neg_inf, jnp.float32)
    inf16 = jnp.full((LANES,), pos_inf, jnp.float32)

    ps16 = ps_v[...]
    res_v[pl.ds(0, LANES)] = neg16
    res_v[pl.ds(LANES, LANES)] = inf16

    def masked_step(j, lo, hi, mv, mp):
        v = vals_v[pl.ds(j * LANES, LANES)]
        pos = base + j * LANES + iota
        inside = (pos >= lo) & (pos < hi)
        v = jnp.where(inside, v, neg_inf)
        upd = v > mv
        mp = jnp.where(upd, j.astype(jnp.float32), mp)
        mv = jnp.where(upd, v, mv)
        return mv, mp

    for s in range(N_SEGS):
        end_s = ps16[s]
        start_s = ps16[s - 1] if s > 0 else jnp.int32(0)
        lo = jnp.maximum(start_s, base)
        hi = jnp.minimum(end_s, base + CHUNK)
        lo_c = jnp.clip(lo - base, 0, CHUNK)
        hi_c = jnp.clip(hi - base, 0, CHUNK)
        j0 = lo_c // LANES
        j1 = (hi_c + (LANES - 1)) // LANES
        jc0 = (lo_c + (LANES - 1)) // LANES
        jc1 = jnp.maximum(hi_c // LANES, jc0)

        @pl.when(j1 > j0)
        def _(s=s, lo=lo, hi=hi, start_s=start_s,
              j0=j0, j1=j1, jc0=jc0, jc1=jc1):
            mv, mp = masked_step(j0, lo, hi, neg16, inf16)

            def body(j, carry):
                mv, mp, jf = carry
                v = vals_v[pl.ds(j * LANES, LANES)]
                upd = v > mv
                mp = jnp.where(upd, jf, mp)
                mv = jnp.where(upd, v, mv)
                return mv, mp, jf + 1.0

            mv, mp, _ = lax.fori_loop(
                jc0, jc1, body, (mv, mp, jc0.astype(jnp.float32)))

            mv, mp = masked_step(jnp.maximum(j1 - 1, j0), lo, hi, mv, mp)

            off = (base - start_s).astype(jnp.float32)
            lp = jnp.where(mv > neg_inf, mp * 16.0 + iota_f + off, pos_inf)

            m = mv
            for k in (8, 4, 2, 1):
                m = jnp.maximum(m, m.at[iota ^ k].get(mode="promise_in_bounds"))
            p = jnp.where(mv == m, lp, inf16)
            for k in (8, 4, 2, 1):
                p = jnp.minimum(p, p.at[iota ^ k].get(mode="promise_in_bounds"))

            lane = iota == s
            res_v[pl.ds(0, LANES)] = jnp.where(lane, m, res_v[pl.ds(0, LANES)])
            res_v[pl.ds(LANES, LANES)] = jnp.where(
                lane, p, res_v[pl.ds(LANES, LANES)])

    pltpu.sync_copy(res_v, stage_hbm.at[sid])
    plsc.subcore_barrier()

    @pl.when(sid == 0)
    def _():
        pltpu.sync_copy(stage_hbm, buf_v)
        gmax = neg16
        for i in range(NUM_SUBCORES):
            gmax = jnp.maximum(gmax, buf_v[i, pl.ds(0, LANES)])
        gpos = inf16
        for i in range(NUM_SUBCORES):
            rv = buf_v[i, pl.ds(0, LANES)]
            rp = buf_v[i, pl.ds(LANES, LANES)]
            gpos = jnp.minimum(gpos, jnp.where(rv == gmax, rp, inf16))
        empty = gmax == neg16
        out_v[...] = jnp.where(empty, jnp.full((LANES,), I32_MAX, jnp.int32),
                               gpos.astype(jnp.int32))
        pltpu.sync_copy(out_v, out_hbm)


@jax.jit
def _jagged_argmax_sc(values, prefix_sum):
    mesh = plsc.VectorSubcoreMesh(
        core_axis_name="c", subcore_axis_name="s",
        num_cores=1, num_subcores=NUM_SUBCORES)
    _, out = pl.kernel(
        _sc_body,
        out_type=[
            jax.ShapeDtypeStruct((NUM_SUBCORES, 2 * LANES), jnp.float32),
            jax.ShapeDtypeStruct((N_SEGS,), jnp.int32),
        ],
        mesh=mesh,
        scratch_types=[
            pltpu.VMEM((CHUNK,), jnp.float32),
            pltpu.VMEM((N_SEGS,), jnp.int32),
            pltpu.VMEM((2 * LANES,), jnp.float32),
            pltpu.VMEM((NUM_SUBCORES, 2 * LANES), jnp.float32),
            pltpu.VMEM((N_SEGS,), jnp.int32),
        ],
    )(values, prefix_sum)
    return out


def kernel(values, prefix_sum):
    out = _jagged_argmax_sc(values, prefix_sum.astype(jnp.int32))
    return out.astype(jnp.int64)


# speedup vs baseline: 1.1704x; 1.1704x over previous
import jax
import jax.numpy as jnp
from jax import lax
from jax.experimental import pallas as pl
from jax.experimental.pallas import tpu as pltpu
from jax.experimental.pallas import tpu_sc as plsc

N_TOKENS = 32768
N_SEGS = 16
NUM_SUBCORES = 16
CHUNK = N_TOKENS // NUM_SUBCORES
LANES = 16
I32_MAX = jnp.iinfo(jnp.int32).max


def _sc_body(values_hbm, ps_hbm, stage_hbm, out_hbm,
             vals_v, ps_v, res_v, buf_v, out_v, ps_s, sem1, sem2):
    sid = lax.axis_index("s")
    base = sid * CHUNK

    cp1 = pltpu.async_copy(values_hbm.at[pl.ds(base, CHUNK)], vals_v, sem1)
    cp2 = pltpu.async_copy(ps_hbm, ps_v, sem2)
    cp2.wait()
    cp1.wait()

    iota = lax.iota(jnp.int32, LANES)
    iota_f = iota.astype(jnp.float32)
    neg_inf = jnp.float32(-jnp.inf)
    pos_inf = jnp.float32(jnp.inf)
    neg16 = jnp.full((LANES,), neg_inf, jnp.float32)
    inf16 = jnp.full((LANES,), pos_inf, jnp.float32)

    ps16 = ps_v[...]
    for i in range(N_SEGS):
        ps_s[i] = ps16[i]
    res_v[pl.ds(0, LANES)] = neg16
    res_v[pl.ds(LANES, LANES)] = inf16

    def masked_step(j, lo, hi, mv, mp):
        v = vals_v[pl.ds(j * LANES, LANES)]
        pos = base + j * LANES + iota
        inside = (pos >= lo) & (pos < hi)
        v = jnp.where(inside, v, neg_inf)
        upd = v > mv
        mp = jnp.where(upd, j.astype(jnp.float32), mp)
        mv = jnp.where(upd, v, mv)
        return mv, mp

    def seg_body(s, start_s):
        end_s = ps_s[s]
        lo = jnp.maximum(start_s, base)
        hi = jnp.minimum(end_s, base + CHUNK)
        lo_c = jnp.clip(lo - base, 0, CHUNK)
        hi_c = jnp.clip(hi - base, 0, CHUNK)
        j0 = lo_c // LANES
        j1 = (hi_c + (LANES - 1)) // LANES
        jc0 = (lo_c + (LANES - 1)) // LANES
        jc1 = jnp.maximum(hi_c // LANES, jc0)

        @pl.when(j1 > j0)
        def _():
            mv, mp = masked_step(j0, lo, hi, neg16, inf16)

            def body(j, carry):
                mv, mp, jf = carry
                v = vals_v[pl.ds(j * LANES, LANES)]
                upd = v > mv
                mp = jnp.where(upd, jf, mp)
                mv = jnp.where(upd, v, mv)
                return mv, mp, jf + 1.0

            mv, mp, _ = lax.fori_loop(
                jc0, jc1, body, (mv, mp, jc0.astype(jnp.float32)))

            mv, mp = masked_step(jnp.maximum(j1 - 1, j0), lo, hi, mv, mp)

            off = (base - start_s).astype(jnp.float32)
            lp = jnp.where(mv > neg_inf, mp * 16.0 + iota_f + off, pos_inf)

            m = mv
            for k in (8, 4, 2, 1):
                m = jnp.maximum(m, m.at[iota ^ k].get(mode="promise_in_bounds"))
            p = jnp.where(mv == m, lp, inf16)
            for k in (8, 4, 2, 1):
                p = jnp.minimum(p, p.at[iota ^ k].get(mode="promise_in_bounds"))

            lane = iota == s
            res_v[pl.ds(0, LANES)] = jnp.where(lane, m, res_v[pl.ds(0, LANES)])
            res_v[pl.ds(LANES, LANES)] = jnp.where(
                lane, p, res_v[pl.ds(LANES, LANES)])

        return end_s

    lax.fori_loop(0, N_SEGS, seg_body, jnp.int32(0))

    pltpu.sync_copy(res_v, stage_hbm.at[sid])
    plsc.subcore_barrier()

    @pl.when(sid == 0)
    def _():
        pltpu.sync_copy(stage_hbm, buf_v)
        gmax = neg16
        for i in range(NUM_SUBCORES):
            gmax = jnp.maximum(gmax, buf_v[i, pl.ds(0, LANES)])
        gpos = inf16
        for i in range(NUM_SUBCORES):
            rv = buf_v[i, pl.ds(0, LANES)]
            rp = buf_v[i, pl.ds(LANES, LANES)]
            gpos = jnp.minimum(gpos, jnp.where(rv == gmax, rp, inf16))
        empty = gmax == neg16
        out_v[...] = jnp.where(empty, jnp.full((LANES,), I32_MAX, jnp.int32),
                               gpos.astype(jnp.int32))
        pltpu.sync_copy(out_v, out_hbm)


@jax.jit
def _jagged_argmax_sc(values, prefix_sum):
    mesh = plsc.VectorSubcoreMesh(
        core_axis_name="c", subcore_axis_name="s",
        num_cores=1, num_subcores=NUM_SUBCORES)
    _, out = pl.kernel(
        _sc_body,
        out_type=[
            jax.ShapeDtypeStruct((NUM_SUBCORES, 2 * LANES), jnp.float32),
            jax.ShapeDtypeStruct((N_SEGS,), jnp.int32),
        ],
        mesh=mesh,
        scratch_types=[
            pltpu.VMEM((CHUNK,), jnp.float32),
            pltpu.VMEM((N_SEGS,), jnp.int32),
            pltpu.VMEM((2 * LANES,), jnp.float32),
            pltpu.VMEM((NUM_SUBCORES, 2 * LANES), jnp.float32),
            pltpu.VMEM((N_SEGS,), jnp.int32),
            pltpu.SMEM((N_SEGS,), jnp.int32),
            pltpu.SemaphoreType.DMA,
            pltpu.SemaphoreType.DMA,
        ],
    )(values, prefix_sum)
    return out


def kernel(values, prefix_sum):
    out = _jagged_argmax_sc(values, prefix_sum.astype(jnp.int32))
    return out.astype(jnp.int64)


# speedup vs baseline: 1.1806x; 1.0087x over previous
import jax
import jax.numpy as jnp
from jax import lax
from jax.experimental import pallas as pl
from jax.experimental.pallas import tpu as pltpu
from jax.experimental.pallas import tpu_sc as plsc

N_TOKENS = 32768
N_SEGS = 16
NUM_SUBCORES = 16
CHUNK = N_TOKENS // NUM_SUBCORES
LANES = 16
I32_MAX = jnp.iinfo(jnp.int32).max


def _sc_body(values_hbm, ps_hbm, stage_hbm, out_hbm,
             vals_v, ps_v, res_v, buf_v, out_v, ps_s, sem1, sem2):
    sid = lax.axis_index("s")
    base = sid * CHUNK

    cp1 = pltpu.async_copy(values_hbm.at[pl.ds(base, CHUNK)], vals_v, sem1)
    cp2 = pltpu.async_copy(ps_hbm, ps_v, sem2)
    cp2.wait()
    cp1.wait()

    iota = lax.iota(jnp.int32, LANES)
    iota_f = iota.astype(jnp.float32)
    neg_inf = jnp.float32(-jnp.inf)
    pos_inf = jnp.float32(jnp.inf)
    neg16 = jnp.full((LANES,), neg_inf, jnp.float32)
    inf16 = jnp.full((LANES,), pos_inf, jnp.float32)

    ps16 = ps_v[...]
    for i in range(N_SEGS):
        ps_s[i] = ps16[i]
    res_v[pl.ds(0, LANES)] = neg16
    res_v[pl.ds(LANES, LANES)] = inf16

    def masked_step(j, lo, hi, mv, mp):
        v = vals_v[pl.ds(j * LANES, LANES)]
        pos = base + j * LANES + iota
        inside = (pos >= lo) & (pos < hi)
        v = jnp.where(inside, v, neg_inf)
        upd = v > mv
        mp = jnp.where(upd, j.astype(jnp.float32), mp)
        mv = jnp.where(upd, v, mv)
        return mv, mp

    def seg_body(s, start_s):
        end_s = ps_s[s]
        lo = jnp.maximum(start_s, base)
        hi = jnp.minimum(end_s, base + CHUNK)
        lo_c = jnp.clip(lo - base, 0, CHUNK)
        hi_c = jnp.clip(hi - base, 0, CHUNK)
        j0 = lo_c // LANES
        j1 = (hi_c + (LANES - 1)) // LANES
        jc0 = (lo_c + (LANES - 1)) // LANES
        jc1 = jnp.maximum(hi_c // LANES, jc0)

        @pl.when(j1 > j0)
        def _():
            mv, mp = masked_step(j0, lo, hi, neg16, inf16)

            def body(j, carry):
                mv, mp, jf = carry
                v = vals_v[pl.ds(j * LANES, LANES)]
                upd = v > mv
                mp = jnp.where(upd, jf, mp)
                mv = jnp.where(upd, v, mv)
                return mv, mp, jf + 1.0

            mv, mp, _ = lax.fori_loop(
                jc0, jc1, body, (mv, mp, jc0.astype(jnp.float32)))

            mv, mp = masked_step(jnp.maximum(j1 - 1, j0), lo, hi, mv, mp)

            off = (base - start_s).astype(jnp.float32)
            lp = jnp.where(mv > neg_inf, mp * 16.0 + iota_f + off, pos_inf)

            m = mv
            for k in (8, 4, 2, 1):
                m = jnp.maximum(m, m.at[iota ^ k].get(mode="promise_in_bounds"))
            p = jnp.where(mv == m, lp, inf16)
            for k in (8, 4, 2, 1):
                p = jnp.minimum(p, p.at[iota ^ k].get(mode="promise_in_bounds"))

            lane = iota == s
            res_v[pl.ds(0, LANES)] = jnp.where(lane, m, res_v[pl.ds(0, LANES)])
            res_v[pl.ds(LANES, LANES)] = jnp.where(
                lane, p, res_v[pl.ds(LANES, LANES)])

        return end_s

    lax.fori_loop(0, N_SEGS, seg_body, jnp.int32(0))

    pltpu.sync_copy(res_v, stage_hbm.at[sid])
    plsc.subcore_barrier()

    @pl.when(sid == 0)
    def _():
        pltpu.sync_copy(stage_hbm, buf_v)

        def merge(i, carry):
            gmax, gpos = carry
            rv = buf_v[i, pl.ds(0, LANES)]
            rp = buf_v[i, pl.ds(LANES, LANES)]
            better = rv > gmax
            tie = rv == gmax
            gpos = jnp.where(better, rp,
                             jnp.where(tie, jnp.minimum(gpos, rp), gpos))
            gmax = jnp.maximum(gmax, rv)
            return gmax, gpos

        gmax, gpos = lax.fori_loop(0, NUM_SUBCORES, merge, (neg16, inf16))
        empty = gmax == neg16
        out_v[...] = jnp.where(empty, jnp.full((LANES,), I32_MAX, jnp.int32),
                               gpos.astype(jnp.int32))
        pltpu.sync_copy(out_v, out_hbm)


@jax.jit
def _jagged_argmax_sc(values, prefix_sum):
    mesh = plsc.VectorSubcoreMesh(
        core_axis_name="c", subcore_axis_name="s",
        num_cores=1, num_subcores=NUM_SUBCORES)
    _, out = pl.kernel(
        _sc_body,
        out_type=[
            jax.ShapeDtypeStruct((NUM_SUBCORES, 2 * LANES), jnp.float32),
            jax.ShapeDtypeStruct((N_SEGS,), jnp.int32),
        ],
        mesh=mesh,
        scratch_types=[
            pltpu.VMEM((CHUNK,), jnp.float32),
            pltpu.VMEM((N_SEGS,), jnp.int32),
            pltpu.VMEM((2 * LANES,), jnp.float32),
            pltpu.VMEM((NUM_SUBCORES, 2 * LANES), jnp.float32),
            pltpu.VMEM((N_SEGS,), jnp.int32),
            pltpu.SMEM((N_SEGS,), jnp.int32),
            pltpu.SemaphoreType.DMA,
            pltpu.SemaphoreType.DMA,
        ],
    )(values, prefix_sum)
    return out


def kernel(values, prefix_sum):
    out = _jagged_argmax_sc(values, prefix_sum.astype(jnp.int32))
    return out.astype(jnp.int64)


# speedup vs baseline: 1.1825x; 1.0016x over previous
import jax
import jax.numpy as jnp
from jax import lax
from jax.experimental import pallas as pl
from jax.experimental.pallas import tpu as pltpu
from jax.experimental.pallas import tpu_sc as plsc

N_TOKENS = 32768
N_SEGS = 16
NUM_SUBCORES = 16
CHUNK = N_TOKENS // NUM_SUBCORES
LANES = 16
I32_MAX = jnp.iinfo(jnp.int32).max


def _sc_body(values_hbm, ps_hbm, out_hbm,
             stage_hbm, vals_v, ps_v, res_v, buf_v, out_v, ps_s, sem1, sem2):
    sid = lax.axis_index("s")
    base = sid * CHUNK

    cp1 = pltpu.async_copy(values_hbm.at[pl.ds(base, CHUNK)], vals_v, sem1)
    cp2 = pltpu.async_copy(ps_hbm, ps_v, sem2)
    cp2.wait()
    cp1.wait()

    iota = lax.iota(jnp.int32, LANES)
    iota_f = iota.astype(jnp.float32)
    neg_inf = jnp.float32(-jnp.inf)
    pos_inf = jnp.float32(jnp.inf)
    neg16 = jnp.full((LANES,), neg_inf, jnp.float32)
    inf16 = jnp.full((LANES,), pos_inf, jnp.float32)

    ps16 = ps_v[...]
    for i in range(N_SEGS):
        ps_s[i] = ps16[i]
    res_v[pl.ds(0, LANES)] = neg16
    res_v[pl.ds(LANES, LANES)] = inf16

    def masked_step(j, lo, hi, mv, mp):
        v = vals_v[pl.ds(j * LANES, LANES)]
        pos = base + j * LANES + iota
        inside = (pos >= lo) & (pos < hi)
        v = jnp.where(inside, v, neg_inf)
        upd = v > mv
        mp = jnp.where(upd, j.astype(jnp.float32), mp)
        mv = jnp.where(upd, v, mv)
        return mv, mp

    def seg_body(s, start_s):
        end_s = ps_s[s]
        lo = jnp.maximum(start_s, base)
        hi = jnp.minimum(end_s, base + CHUNK)
        lo_c = jnp.clip(lo - base, 0, CHUNK)
        hi_c = jnp.clip(hi - base, 0, CHUNK)
        j0 = lo_c // LANES
        j1 = (hi_c + (LANES - 1)) // LANES
        jc0 = (lo_c + (LANES - 1)) // LANES
        jc1 = jnp.maximum(hi_c // LANES, jc0)

        @pl.when(j1 > j0)
        def _():
            mv, mp = masked_step(j0, lo, hi, neg16, inf16)

            def body(j, carry):
                mv, mp, jf = carry
                v = vals_v[pl.ds(j * LANES, LANES)]
                upd = v > mv
                mp = jnp.where(upd, jf, mp)
                mv = jnp.where(upd, v, mv)
                return mv, mp, jf + 1.0

            mv, mp, _ = lax.fori_loop(
                jc0, jc1, body, (mv, mp, jc0.astype(jnp.float32)))

            mv, mp = masked_step(jnp.maximum(j1 - 1, j0), lo, hi, mv, mp)

            off = (base - start_s).astype(jnp.float32)
            lp = jnp.where(mv > neg_inf, mp * 16.0 + iota_f + off, pos_inf)

            m = mv
            for k in (8, 4, 2, 1):
                m = jnp.maximum(m, m.at[iota ^ k].get(mode="promise_in_bounds"))
            p = jnp.where(mv == m, lp, inf16)
            for k in (8, 4, 2, 1):
                p = jnp.minimum(p, p.at[iota ^ k].get(mode="promise_in_bounds"))

            lane = iota == s
            res_v[pl.ds(0, LANES)] = jnp.where(lane, m, res_v[pl.ds(0, LANES)])
            res_v[pl.ds(LANES, LANES)] = jnp.where(
                lane, p, res_v[pl.ds(LANES, LANES)])

        return end_s

    lax.fori_loop(0, N_SEGS, seg_body, jnp.int32(0))

    pltpu.sync_copy(res_v, stage_hbm.at[sid])
    plsc.subcore_barrier()

    @pl.when(sid == 0)
    def _():
        pltpu.sync_copy(stage_hbm, buf_v)

        def merge(i, carry):
            gmax, gpos = carry
            rv = buf_v[i, pl.ds(0, LANES)]
            rp = buf_v[i, pl.ds(LANES, LANES)]
            better = rv > gmax
            tie = rv == gmax
            gpos = jnp.where(better, rp,
                             jnp.where(tie, jnp.minimum(gpos, rp), gpos))
            gmax = jnp.maximum(gmax, rv)
            return gmax, gpos

        gmax, gpos = lax.fori_loop(0, NUM_SUBCORES, merge, (neg16, inf16))
        empty = gmax == neg16
        out_v[...] = jnp.where(empty, jnp.full((LANES,), I32_MAX, jnp.int32),
                               gpos.astype(jnp.int32))
        pltpu.sync_copy(out_v, out_hbm)


@jax.jit
def _jagged_argmax_sc(values, prefix_sum):
    mesh = plsc.VectorSubcoreMesh(
        core_axis_name="c", subcore_axis_name="s",
        num_cores=1, num_subcores=NUM_SUBCORES)
    out = pl.kernel(
        _sc_body,
        out_type=jax.ShapeDtypeStruct((N_SEGS,), jnp.int32),
        mesh=mesh,
        scratch_types=[
            pltpu.HBM((NUM_SUBCORES, 2 * LANES), jnp.float32),
            pltpu.VMEM((CHUNK,), jnp.float32),
            pltpu.VMEM((N_SEGS,), jnp.int32),
            pltpu.VMEM((2 * LANES,), jnp.float32),
            pltpu.VMEM((NUM_SUBCORES, 2 * LANES), jnp.float32),
            pltpu.VMEM((N_SEGS,), jnp.int32),
            pltpu.SMEM((N_SEGS,), jnp.int32),
            pltpu.SemaphoreType.DMA,
            pltpu.SemaphoreType.DMA,
        ],
    )(values, prefix_sum)
    return out


def kernel(values, prefix_sum):
    out = _jagged_argmax_sc(values, prefix_sum.astype(jnp.int32))
    return out.astype(jnp.int64)


# speedup vs baseline: 1.1843x; 1.0015x over previous
import jax
import jax.numpy as jnp
from jax import lax
from jax.experimental import pallas as pl
from jax.experimental.pallas import tpu as pltpu
from jax.experimental.pallas import tpu_sc as plsc

N_TOKENS = 32768
N_SEGS = 16
NUM_SUBCORES = 16
CHUNK = N_TOKENS // NUM_SUBCORES
LANES = 16
I32_MAX = jnp.iinfo(jnp.int32).max


def _sc_body(values_hbm, ps_hbm, out_hbm,
             stage_hbm, vals_v, ps_v, res_v, buf_v, out_v, ps_s, sem1, sem2):
    sid = lax.axis_index("s")
    base = sid * CHUNK

    cp1 = pltpu.async_copy(values_hbm.at[pl.ds(base, CHUNK)], vals_v, sem1)
    cp2 = pltpu.async_copy(ps_hbm, ps_v, sem2)
    cp2.wait()
    cp1.wait()

    iota = lax.iota(jnp.int32, LANES)
    iota_f = iota.astype(jnp.float32)
    neg_inf = jnp.float32(-jnp.inf)
    pos_inf = jnp.float32(jnp.inf)
    neg16 = jnp.full((LANES,), neg_inf, jnp.float32)
    inf16 = jnp.full((LANES,), pos_inf, jnp.float32)

    ps16 = ps_v[...]
    for i in range(N_SEGS):
        ps_s[i] = ps16[i]
    res_v[pl.ds(0, LANES)] = neg16
    res_v[pl.ds(LANES, LANES)] = inf16

    def masked_step(j, lo, hi, mv, mp):
        v = vals_v[pl.ds(j * LANES, LANES)]
        pos = base + j * LANES + iota
        inside = (pos >= lo) & (pos < hi)
        v = jnp.where(inside, v, neg_inf)
        upd = v > mv
        mp = jnp.where(upd, j.astype(jnp.float32), mp)
        mv = jnp.where(upd, v, mv)
        return mv, mp

    def seg_body(s, start_s):
        end_s = ps_s[s]
        lo = jnp.maximum(start_s, base)
        hi = jnp.minimum(end_s, base + CHUNK)
        lo_c = jnp.clip(lo - base, 0, CHUNK)
        hi_c = jnp.clip(hi - base, 0, CHUNK)
        j0 = lo_c // LANES
        j1 = (hi_c + (LANES - 1)) // LANES

        @pl.when(j1 > j0)
        def _():
            def body(j, carry):
                return masked_step(j, lo, hi, *carry)

            mv, mp = lax.fori_loop(j0, j1, body, (neg16, inf16))

            off = (base - start_s).astype(jnp.float32)
            lp = jnp.where(mv > neg_inf, mp * 16.0 + iota_f + off, pos_inf)

            m = mv
            for k in (8, 4, 2, 1):
                m = jnp.maximum(m, m.at[iota ^ k].get(mode="promise_in_bounds"))
            p = jnp.where(mv == m, lp, inf16)
            for k in (8, 4, 2, 1):
                p = jnp.minimum(p, p.at[iota ^ k].get(mode="promise_in_bounds"))

            lane = iota == s
            res_v[pl.ds(0, LANES)] = jnp.where(lane, m, res_v[pl.ds(0, LANES)])
            res_v[pl.ds(LANES, LANES)] = jnp.where(
                lane, p, res_v[pl.ds(LANES, LANES)])

        return end_s

    lax.fori_loop(0, N_SEGS, seg_body, jnp.int32(0))

    pltpu.sync_copy(res_v, stage_hbm.at[sid])
    plsc.subcore_barrier()

    @pl.when(sid == 0)
    def _():
        pltpu.sync_copy(stage_hbm, buf_v)

        def merge(i, carry):
            gmax, gpos = carry
            rv = buf_v[i, pl.ds(0, LANES)]
            rp = buf_v[i, pl.ds(LANES, LANES)]
            better = rv > gmax
            tie = rv == gmax
            gpos = jnp.where(better, rp,
                             jnp.where(tie, jnp.minimum(gpos, rp), gpos))
            gmax = jnp.maximum(gmax, rv)
            return gmax, gpos

        gmax, gpos = lax.fori_loop(0, NUM_SUBCORES, merge, (neg16, inf16))
        empty = gmax == neg16
        out_v[...] = jnp.where(empty, jnp.full((LANES,), I32_MAX, jnp.int32),
                               gpos.astype(jnp.int32))
        pltpu.sync_copy(out_v, out_hbm)


@jax.jit
def _jagged_argmax_sc(values, prefix_sum):
    mesh = plsc.VectorSubcoreMesh(
        core_axis_name="c", subcore_axis_name="s",
        num_cores=1, num_subcores=NUM_SUBCORES)
    out = pl.kernel(
        _sc_body,
        out_type=jax.ShapeDtypeStruct((N_SEGS,), jnp.int32),
        mesh=mesh,
        scratch_types=[
            pltpu.HBM((NUM_SUBCORES, 2 * LANES), jnp.float32),
            pltpu.VMEM((CHUNK,), jnp.float32),
            pltpu.VMEM((N_SEGS,), jnp.int32),
            pltpu.VMEM((2 * LANES,), jnp.float32),
            pltpu.VMEM((NUM_SUBCORES, 2 * LANES), jnp.float32),
            pltpu.VMEM((N_SEGS,), jnp.int32),
            pltpu.SMEM((N_SEGS,), jnp.int32),
            pltpu.SemaphoreType.DMA,
            pltpu.SemaphoreType.DMA,
        ],
    )(values, prefix_sum)
    return out


def kernel(values, prefix_sum):
    out = _jagged_argmax_sc(values, prefix_sum.astype(jnp.int32))
    return out.astype(jnp.int64)
